# baseline (speedup 1.0000x reference)
import functools

import jax
import jax.numpy as jnp
from jax.experimental import pallas as pl
from jax.experimental.pallas import tpu as pltpu

N = 5000
NP = 5120
RB = 128
GRID = NP // RB
KL = 152
KP = 8
HID = 64
K = 256
IM_W, IM_H = 1024.0, 1024.0

_DN = (((1,), (0,)), ((), ()))


def _relness_body(logits_ref, pos_ref, boxes_ref, boxesT_ref,
                  logitsT_ref, posT_ref,
                  W_subj_ref, b_subj_ref, W_objT_ref, b_obj_ref,
                  W_pos_s_ref, b_pos_s_ref, W_pos_oT_ref, b_pos_o_ref,
                  out_ref, xobjT_ref, poT_ref):
    i = pl.program_id(0)

    @pl.when(i == 0)
    def _():
        xobjT_ref[...] = jax.lax.dot_general(
            W_objT_ref[...], logitsT_ref[...], _DN) + b_obj_ref[...]
        poT_ref[...] = jax.lax.dot_general(
            W_pos_oT_ref[...], posT_ref[...], _DN) + b_pos_o_ref[...]

    x_subj = jax.lax.dot_general(
        logits_ref[...], W_subj_ref[...], _DN) + b_subj_ref[...]
    p_s = jax.lax.dot_general(
        pos_ref[...], W_pos_s_ref[...], _DN) + b_pos_s_ref[...]
    scores = jax.lax.dot_general(x_subj, xobjT_ref[...], _DN)
    scores = scores + jax.lax.dot_general(p_s, poT_ref[...], _DN)
    relness = jax.nn.sigmoid(scores)

    rx1 = boxes_ref[:, 0:1]
    ry1 = boxes_ref[:, 1:2]
    rx2 = boxes_ref[:, 2:3]
    ry2 = boxes_ref[:, 3:4]
    cx1 = boxesT_ref[0:1, :]
    cy1 = boxesT_ref[1:2, :]
    cx2 = boxesT_ref[2:3, :]
    cy2 = boxesT_ref[3:4, :]
    area_r = (rx2 - rx1) * (ry2 - ry1)
    area_c = (cx2 - cx1) * (cy2 - cy1)
    iw = jnp.clip(jnp.minimum(rx2, cx2) - jnp.maximum(rx1, cx1), 0.0)
    ih = jnp.clip(jnp.minimum(ry2, cy2) - jnp.maximum(ry1, cy1), 0.0)
    inter = iw * ih
    union = area_r + area_c - inter
    iou = inter / (union + 1e-9)
    rows = i * RB + jax.lax.broadcasted_iota(jnp.int32, (RB, NP), 0)
    cols = jax.lax.broadcasted_iota(jnp.int32, (RB, NP), 1)
    keep = (iou > 0.0) & (rows != cols) & (rows < N) & (cols < N)
    out_ref[...] = jnp.where(keep, relness, -1.0)


def _pad_to(x, shape):
    pads = [(0, t - s) for s, t in zip(x.shape, shape)]
    return jnp.pad(x, pads)


@functools.partial(jax.jit, static_argnums=())
def _masked_relness(boxes, logits, W_subj, b_subj, W_obj, b_obj,
                    W_pos_s, b_pos_s, W_pos_o, b_pos_o):
    f32 = jnp.float32
    scale = jnp.array([IM_W, IM_H, IM_W, IM_H], dtype=boxes.dtype)
    pos = boxes / scale

    logits_p = _pad_to(logits, (NP, KL))
    logitsT_p = _pad_to(logits.T, (KL, NP))
    pos_p = _pad_to(pos, (NP, KP))
    posT_p = _pad_to(pos.T, (KP, NP))
    boxes_p = _pad_to(boxes, (NP, KP))
    boxesT_p = _pad_to(boxes.T, (KP, NP))
    W_subj_p = _pad_to(W_subj, (KL, HID))
    W_objT_p = _pad_to(W_obj.T, (HID, KL))
    W_pos_s_p = _pad_to(W_pos_s, (KP, HID))
    W_pos_oT_p = _pad_to(W_pos_o.T, (HID, KP))
    b_subj_2d = b_subj[None, :]
    b_pos_s_2d = b_pos_s[None, :]
    b_obj_bc = jnp.broadcast_to(b_obj[:, None], (HID, NP))
    b_pos_o_bc = jnp.broadcast_to(b_pos_o[:, None], (HID, NP))

    full = lambda shape: pl.BlockSpec(shape, lambda i: (0, 0))
    rowblk = lambda w: pl.BlockSpec((RB, w), lambda i: (i, 0))

    masked = pl.pallas_call(
        _relness_body,
        grid=(GRID,),
        in_specs=[
            rowblk(KL),
            rowblk(KP),
            rowblk(KP),
            full((KP, NP)),
            full((KL, NP)),
            full((KP, NP)),
            full((KL, HID)),
            full((1, HID)),
            full((HID, KL)),
            full((HID, NP)),
            full((KP, HID)),
            full((1, HID)),
            full((HID, KP)),
            full((HID, NP)),
        ],
        out_specs=pl.BlockSpec((RB, NP), lambda i: (i, 0)),
        out_shape=jax.ShapeDtypeStruct((NP, NP), f32),
        scratch_shapes=[
            pltpu.VMEM((HID, NP), f32),
            pltpu.VMEM((HID, NP), f32),
        ],
    )(logits_p, pos_p, boxes_p, boxesT_p, logitsT_p, posT_p,
      W_subj_p, b_subj_2d, W_objT_p, b_obj_bc,
      W_pos_s_p, b_pos_s_2d, W_pos_oT_p, b_pos_o_bc)
    return masked


def kernel(boxes, logits, W_subj, b_subj, W_obj, b_obj,
           W_pos_s, b_pos_s, W_pos_o, b_pos_o):
    masked = _masked_relness(boxes, logits, W_subj, b_subj, W_obj, b_obj,
                             W_pos_s, b_pos_s, W_pos_o, b_pos_o)
    flat = masked[:N, :N].reshape(-1)
    top_vals, top_idx = jax.lax.top_k(flat, K)
    idx_subj = top_idx // N
    idx_obj = top_idx % N
    pair_boxes = jnp.concatenate([jnp.take(boxes, idx_subj, axis=0),
                                  jnp.take(boxes, idx_obj, axis=0)], axis=1)
    idx_pairs = jnp.stack([idx_subj, idx_obj], axis=1)
    return top_vals, pair_boxes, idx_pairs
